# baseline (speedup 1.0000x reference)
import functools

import jax
import jax.numpy as jnp
from jax import lax
from jax.experimental import pallas as pl
from jax.experimental.pallas import tpu as pltpu
from jax.experimental.pallas import tpu_sc as plsc

_NC = 2
_NS = 16
_SEQ = 2048
_D = 64
_KPW = 8
_JW = 1024
_NKG = _D // _KPW
_NJW = _SEQ // _JW
_ROWS_PER_CORE = _SEQ // _NC
_NBUF = 4


def _gather_t(idx, table_t):
    mesh = plsc.VectorSubcoreMesh(core_axis_name="c", subcore_axis_name="s")

    @functools.partial(
        pl.kernel,
        out_type=jax.ShapeDtypeStruct((_SEQ, _D, _SEQ), jnp.float32),
        mesh=mesh,
        scratch_types=(
            [pltpu.VMEM((1, 4097), jnp.float32)] * _KPW
            + [pltpu.VMEM((_JW,), jnp.int32)] * _NBUF
            + [pltpu.VMEM((1, _KPW, _JW), jnp.float32)] * _NBUF
            + [pltpu.SemaphoreType.DMA] * (2 * _NBUF)
        ),
        compiler_params=pltpu.CompilerParams(needs_layout_passes=False),
    )
    def k(idx_hbm, tab_hbm, out_hbm, *bufs):
        tabv = bufs[0:_KPW]
        bufs = bufs[_KPW:]
        idxv = bufs[0:_NBUF]
        outv = bufs[_NBUF:2 * _NBUF]
        sem_idx = bufs[2 * _NBUF:3 * _NBUF]
        sem_out = bufs[3 * _NBUF:4 * _NBUF]
        c = lax.axis_index("c")
        s = lax.axis_index("s")
        k0 = (s % _NKG) * _KPW
        j0 = (s // _NKG) * _JW
        row0 = c * _ROWS_PER_CORE

        for kr in range(_KPW):
            pltpu.sync_copy(tab_hbm.at[pl.ds(k0 + kr, 1)], tabv[kr])

        def issue_idx(i, b):
            pltpu.async_copy(
                idx_hbm.at[pl.ds((row0 + i) * _SEQ + j0, _JW)],
                idxv[b], sem_idx[b])

        def wait_idx(b):
            pltpu.make_async_copy(
                idx_hbm.at[pl.ds(0, _JW)],
                idxv[b], sem_idx[b]).wait()

        def issue_store(i, b):
            pltpu.async_copy(
                outv[b],
                out_hbm.at[pl.ds(row0 + i, 1), pl.ds(k0, _KPW), pl.ds(j0, _JW)],
                sem_out[b])

        def wait_store(b):
            pltpu.make_async_copy(
                outv[b],
                out_hbm.at[pl.ds(0, 1), pl.ds(k0, _KPW), pl.ds(j0, _JW)],
                sem_out[b]).wait()

        def compute(b):
            @plsc.parallel_loop(0, _JW // 16, unroll=6)
            def _(j16):
                iv = idxv[b][pl.ds(j16 * 16, 16)]
                zv = jnp.zeros((16,), jnp.int32)
                vals = [plsc.load_gather(tabv[kr], [zv, iv])
                        for kr in range(_KPW)]
                for kr in range(_KPW):
                    outv[b][0, kr, pl.ds(j16 * 16, 16)] = vals[kr]

        for b in range(_NBUF):
            issue_idx(b, b)

        def body(i, b):
            wait_idx(b)

            @pl.when(i >= _NBUF)
            def _():
                wait_store(b)

            compute(b)
            issue_store(i, b)

            @pl.when(i + _NBUF < _ROWS_PER_CORE)
            def _():
                issue_idx(i + _NBUF, b)

        def outer(g, carry):
            for b in range(_NBUF):
                body(g * _NBUF + b, b)
            return carry

        lax.fori_loop(0, _ROWS_PER_CORE // _NBUF, outer, 0)
        for b in range(_NBUF):
            wait_store(b)

    return k(idx, table_t)


def kernel(input, embeddings):
    table_t = jnp.swapaxes(embeddings, 0, 1)
    out = _gather_t(input.reshape(-1).astype(jnp.int32), table_t)
    return jnp.swapaxes(out, 1, 2)


# speedup vs baseline: 1.0308x; 1.0308x over previous
import functools

import jax
import jax.numpy as jnp
from jax import lax
from jax.experimental import pallas as pl
from jax.experimental.pallas import tpu as pltpu
from jax.experimental.pallas import tpu_sc as plsc

_NC = 2
_NS = 16
_SEQ = 2048
_D = 64
_KPW = 8
_JW = 1024
_NKG = _D // _KPW
_NJW = _SEQ // _JW
_ROWS_PER_CORE = _SEQ // _NC
_NBUF = 4


def _gather_t(idx, table_t):
    mesh = plsc.VectorSubcoreMesh(core_axis_name="c", subcore_axis_name="s")

    @functools.partial(
        pl.kernel,
        out_type=jax.ShapeDtypeStruct((_SEQ, _D, _SEQ), jnp.float32),
        mesh=mesh,
        scratch_types=(
            [pltpu.VMEM((1, 4097), jnp.float32)] * _KPW
            + [pltpu.VMEM((_JW,), jnp.int32)] * _NBUF
            + [pltpu.VMEM((1, _KPW, _JW), jnp.float32)] * _NBUF
            + [pltpu.SemaphoreType.DMA] * (2 * _NBUF)
        ),
        compiler_params=pltpu.CompilerParams(needs_layout_passes=False),
    )
    def k(idx_hbm, tab_hbm, out_hbm, *bufs):
        tabv = bufs[0:_KPW]
        bufs = bufs[_KPW:]
        idxv = bufs[0:_NBUF]
        outv = bufs[_NBUF:2 * _NBUF]
        sem_idx = bufs[2 * _NBUF:3 * _NBUF]
        sem_out = bufs[3 * _NBUF:4 * _NBUF]
        c = lax.axis_index("c")
        s = lax.axis_index("s")
        k0 = (s % _NKG) * _KPW
        j0 = (s // _NKG) * _JW
        row0 = c * _ROWS_PER_CORE

        for kr in range(_KPW):
            pltpu.sync_copy(tab_hbm.at[pl.ds(k0 + kr, 1)], tabv[kr])

        def issue_idx(i, b):
            pltpu.async_copy(
                idx_hbm.at[pl.ds((row0 + i) * _SEQ + j0, _JW)],
                idxv[b], sem_idx[b])

        def wait_idx(b):
            pltpu.make_async_copy(
                idx_hbm.at[pl.ds(0, _JW)],
                idxv[b], sem_idx[b]).wait()

        def issue_store(i, b):
            pltpu.async_copy(
                outv[b],
                out_hbm.at[pl.ds(row0 + i, 1), pl.ds(k0, _KPW), pl.ds(j0, _JW)],
                sem_out[b])

        def wait_store(b):
            pltpu.make_async_copy(
                outv[b],
                out_hbm.at[pl.ds(0, 1), pl.ds(k0, _KPW), pl.ds(j0, _JW)],
                sem_out[b]).wait()

        def compute(b):
            @plsc.parallel_loop(0, _JW // 16, unroll=5)
            def _(j16):
                iv = idxv[b][pl.ds(j16 * 16, 16)]
                zv = jnp.zeros((16,), jnp.int32)
                vals = [plsc.load_gather(tabv[kr], [zv, iv])
                        for kr in range(_KPW)]
                for kr in range(_KPW):
                    outv[b][0, kr, pl.ds(j16 * 16, 16)] = vals[kr]

        for b in range(_NBUF):
            issue_idx(b, b)

        def body(i, b):
            wait_idx(b)

            @pl.when(i >= _NBUF)
            def _():
                wait_store(b)

            compute(b)
            issue_store(i, b)

            @pl.when(i + _NBUF < _ROWS_PER_CORE)
            def _():
                issue_idx(i + _NBUF, b)

        def outer(g, carry):
            for b in range(_NBUF):
                body(g * _NBUF + b, b)
            return carry

        lax.fori_loop(0, _ROWS_PER_CORE // _NBUF, outer, 0)
        for b in range(_NBUF):
            wait_store(b)

    return k(idx, table_t)


def kernel(input, embeddings):
    table_t = jnp.swapaxes(embeddings, 0, 1)
    out = _gather_t(input.reshape(-1).astype(jnp.int32), table_t)
    return jnp.swapaxes(out, 1, 2)


# speedup vs baseline: 1.2532x; 1.2157x over previous
import functools

import jax
import jax.numpy as jnp
from jax import lax
from jax.experimental import pallas as pl
from jax.experimental.pallas import tpu as pltpu
from jax.experimental.pallas import tpu_sc as plsc

_NC = 2
_NS = 16
_SEQ = 2048
_D = 64
_KPW = 8
_JW = 1024
_NKG = _D // _KPW
_NJW = _SEQ // _JW
_ROWS_PER_CORE = _SEQ // _NC
_NBUF = 4


def _gather_t(idx, table_t):
    mesh = plsc.VectorSubcoreMesh(core_axis_name="c", subcore_axis_name="s")

    @functools.partial(
        pl.kernel,
        out_type=jax.ShapeDtypeStruct((_SEQ, _D, _SEQ), jnp.float32),
        mesh=mesh,
        scratch_types=(
            [pltpu.VMEM((1, 4097), jnp.float32)] * _KPW
            + [pltpu.VMEM((_JW,), jnp.int32)] * _NBUF
            + [pltpu.VMEM((1, _KPW, _JW), jnp.float32)] * _NBUF
            + [pltpu.SemaphoreType.DMA] * (2 * _NBUF)
        ),
        compiler_params=pltpu.CompilerParams(needs_layout_passes=False),
    )
    def k(idx_hbm, tab_hbm, out_hbm, *bufs):
        tabv = bufs[0:_KPW]
        bufs = bufs[_KPW:]
        idxv = bufs[0:_NBUF]
        outv = bufs[_NBUF:2 * _NBUF]
        sem_idx = bufs[2 * _NBUF:3 * _NBUF]
        sem_out = bufs[3 * _NBUF:4 * _NBUF]
        c = lax.axis_index("c")
        s = lax.axis_index("s")
        k0 = (s % _NKG) * _KPW
        j0 = (s // _NKG) * _JW
        row0 = c * _ROWS_PER_CORE

        for kr in range(_KPW):
            pltpu.sync_copy(tab_hbm.at[pl.ds(k0 + kr, 1)], tabv[kr])

        def issue_idx(i, b):
            pltpu.async_copy(
                idx_hbm.at[pl.ds((row0 + i) * _SEQ + j0, _JW)],
                idxv[b], sem_idx[b])

        def wait_idx(b):
            pltpu.make_async_copy(
                idx_hbm.at[pl.ds(0, _JW)],
                idxv[b], sem_idx[b]).wait()

        def issue_store(i, b):
            pltpu.async_copy(
                outv[b],
                out_hbm.at[pl.ds(row0 + i, 1), pl.ds(k0, _KPW), pl.ds(j0, _JW)],
                sem_out[b])

        def wait_store(b):
            pltpu.make_async_copy(
                outv[b],
                out_hbm.at[pl.ds(0, 1), pl.ds(k0, _KPW), pl.ds(j0, _JW)],
                sem_out[b]).wait()

        def compute(b):
            @plsc.parallel_loop(0, _JW // 16, unroll=4)
            def _(j16):
                iv = idxv[b][pl.ds(j16 * 16, 16)]
                zv = jnp.zeros((16,), jnp.int32)
                vals = [plsc.load_gather(tabv[kr], [zv, iv])
                        for kr in range(_KPW)]
                for kr in range(_KPW):
                    outv[b][0, kr, pl.ds(j16 * 16, 16)] = vals[kr]

        for b in range(_NBUF):
            issue_idx(b, b)

        def body(i, b):
            wait_idx(b)

            @pl.when(i >= _NBUF)
            def _():
                wait_store(b)

            compute(b)
            issue_store(i, b)

            @pl.when(i + _NBUF < _ROWS_PER_CORE)
            def _():
                issue_idx(i + _NBUF, b)

        def outer(g, carry):
            for b in range(_NBUF):
                body(g * _NBUF + b, b)
            return carry

        lax.fori_loop(0, _ROWS_PER_CORE // _NBUF, outer, 0)
        for b in range(_NBUF):
            wait_store(b)

    return k(idx, table_t)


def kernel(input, embeddings):
    table_t = jnp.swapaxes(embeddings, 0, 1)
    out = _gather_t(input.reshape(-1).astype(jnp.int32), table_t)
    return jnp.swapaxes(out, 1, 2)
